# baseline (speedup 1.0000x reference)
import functools

import jax
import jax.numpy as jnp
from jax import lax
from jax.experimental import pallas as pl
from jax.experimental.pallas import tpu as pltpu
from jax.experimental.pallas import tpu_sc as plsc

_D = 64
_DP = 128
_NW = 32
_CHUNK = 200
_NBUF = 2


@functools.lru_cache(maxsize=None)
def _build(B):
    b_per_w = B // _NW
    n_chunks = b_per_w // _CHUNK
    mesh = plsc.VectorSubcoreMesh(core_axis_name="c", subcore_axis_name="s")

    @functools.partial(
        pl.kernel,
        mesh=mesh,
        compiler_params=pltpu.CompilerParams(
            use_tc_tiling_on_sc=True, needs_layout_passes=False),
        out_type=jax.ShapeDtypeStruct((B, _D), jnp.float32),
        scratch_types=[
            pltpu.VMEM((b_per_w,), jnp.int32),
            pltpu.VMEM((b_per_w,), jnp.int32),
            pltpu.VMEM((b_per_w,), jnp.int32),
            pltpu.VMEM((_NBUF, _CHUNK, _DP), jnp.float32),
            pltpu.VMEM((_NBUF, _CHUNK, _D), jnp.float32),
            pltpu.SemaphoreType.DMA,
            pltpu.SemaphoreType.DMA,
            pltpu.SemaphoreType.DMA,
            pltpu.SemaphoreType.DMA,
        ],
    )
    def emb(idx_hbm, table_hbm, out_hbm, idx_v, pidx_v, par_v, rows_v,
            half_v, g0, g1, o0, o1):
        gsem = (g0, g1)
        osem = (o0, o1)
        wid = lax.axis_index("s") * 2 + lax.axis_index("c")
        base = wid * b_per_w
        pltpu.sync_copy(idx_hbm.at[pl.ds(base, b_per_w)], idx_v)

        def shift_body(i, _):
            v = idx_v[pl.ds(i * 16, 16)]
            pidx_v[pl.ds(i * 16, 16)] = jax.lax.shift_right_logical(v, 1)
            par_v[pl.ds(i * 16, 16)] = jax.lax.bitwise_and(v, 1)
            return _
        lax.fori_loop(0, b_per_w // 16, shift_body, 0)

        def extract_chunk(b, c):
            def row_body(i, _):
                splat = jnp.full((16,), c * _CHUNK + i, jnp.int32)
                p = plsc.load_gather(par_v, [splat])
                m = p > 0
                for k in range(_D // 16):
                    lo = rows_v[b, i, pl.ds(k * 16, 16)]
                    hi = rows_v[b, i, pl.ds(_D + k * 16, 16)]
                    half_v[b, i, pl.ds(k * 16, 16)] = jnp.where(m, hi, lo)
                return _
            lax.fori_loop(0, _CHUNK, row_body, 0)

        gathers = [None] * _NBUF
        outs = [None] * _NBUF
        for i in range(n_chunks):
            b = i % _NBUF
            if outs[b] is not None:
                outs[b].wait()
            gathers[b] = pltpu.async_copy(
                table_hbm.at[pidx_v.at[pl.ds(i * _CHUNK, _CHUNK)]],
                rows_v.at[b], gsem[b])
            if i > 0:
                pb = (i - 1) % _NBUF
                gathers[pb].wait()
                extract_chunk(pb, i - 1)
                outs[pb] = pltpu.async_copy(
                    half_v.at[pb],
                    out_hbm.at[pl.ds(base + (i - 1) * _CHUNK, _CHUNK)],
                    osem[pb])
        last = n_chunks - 1
        lb = last % _NBUF
        gathers[lb].wait()
        extract_chunk(lb, last)
        outs[lb] = pltpu.async_copy(
            half_v.at[lb],
            out_hbm.at[pl.ds(base + last * _CHUNK, _CHUNK)],
            osem[lb])
        for b in range(_NBUF):
            if outs[b] is not None:
                outs[b].wait()

    return emb


def kernel(input_ids, table):
    bt, h = input_ids.shape
    flat = input_ids.T.reshape(bt * h).astype(jnp.int32)
    t2 = table.reshape(table.shape[0] // 2, _DP)
    out = _build(bt * h)(flat, t2)
    return out.reshape(h, bt, _D).transpose(1, 0, 2)


# speedup vs baseline: 1.1781x; 1.1781x over previous
import functools

import jax
import jax.numpy as jnp
from jax import lax
from jax.experimental import pallas as pl
from jax.experimental.pallas import tpu as pltpu
from jax.experimental.pallas import tpu_sc as plsc

_D = 64
_DP = 128
_NW = 32
_CHUNK = 400
_NBUF = 2


@functools.lru_cache(maxsize=None)
def _build(B):
    b_per_w = B // _NW
    n_chunks = b_per_w // _CHUNK
    mesh = plsc.VectorSubcoreMesh(core_axis_name="c", subcore_axis_name="s")

    @functools.partial(
        pl.kernel,
        mesh=mesh,
        compiler_params=pltpu.CompilerParams(use_tc_tiling_on_sc=True),
        out_type=jax.ShapeDtypeStruct((B, _DP), jnp.float32),
        scratch_types=[
            pltpu.VMEM((b_per_w,), jnp.int32),
            pltpu.VMEM((_NBUF, _CHUNK, _DP), jnp.float32),
            pltpu.SemaphoreType.DMA,
            pltpu.SemaphoreType.DMA,
            pltpu.SemaphoreType.DMA,
            pltpu.SemaphoreType.DMA,
        ],
    )
    def emb(idx_hbm, table_hbm, out_hbm, idx_v, rows_v, g0, g1, o0, o1):
        gsem = (g0, g1)
        osem = (o0, o1)
        wid = lax.axis_index("s") * 2 + lax.axis_index("c")
        base = wid * b_per_w
        pltpu.sync_copy(idx_hbm.at[pl.ds(base, b_per_w)], idx_v)

        gathers = [None] * _NBUF
        outs = [None] * _NBUF
        for i in range(n_chunks):
            b = i % _NBUF
            if outs[b] is not None:
                outs[b].wait()
            gathers[b] = pltpu.async_copy(
                table_hbm.at[idx_v.at[pl.ds(i * _CHUNK, _CHUNK)]],
                rows_v.at[b], gsem[b])
            if i > 0:
                pb = (i - 1) % _NBUF
                gathers[pb].wait()
                outs[pb] = pltpu.async_copy(
                    rows_v.at[pb],
                    out_hbm.at[pl.ds(base + (i - 1) * _CHUNK, _CHUNK)],
                    osem[pb])
        last = n_chunks - 1
        lb = last % _NBUF
        gathers[lb].wait()
        outs[lb] = pltpu.async_copy(
            rows_v.at[lb],
            out_hbm.at[pl.ds(base + last * _CHUNK, _CHUNK)],
            osem[lb])
        for b in range(_NBUF):
            if outs[b] is not None:
                outs[b].wait()

    return emb


def kernel(input_ids, table):
    bt, h = input_ids.shape
    flat = input_ids.T.reshape(bt * h).astype(jnp.int32)
    tpad = jnp.pad(table, ((0, 0), (0, _DP - _D)))
    out = _build(bt * h)(flat, tpad)
    return out[:, :_D].reshape(h, bt, _D).transpose(1, 0, 2)


# speedup vs baseline: 1.6560x; 1.4057x over previous
import functools

import jax
import jax.numpy as jnp
from jax import lax
from jax.experimental import pallas as pl
from jax.experimental.pallas import tpu as pltpu
from jax.experimental.pallas import tpu_sc as plsc

_D = 64
_DP = 128
_NW = 32
_CHUNK = 400
_NBUF = 2
_TBLK = 4096

_mesh = plsc.VectorSubcoreMesh(core_axis_name="c", subcore_axis_name="s")


@functools.lru_cache(maxsize=None)
def _build_tr(V):
    n_blk = (V + _TBLK - 1) // _TBLK

    def tk(t_ref, o_ref):
        o_ref[:, :_D] = t_ref[...].T

    return pl.pallas_call(
        tk,
        grid=(n_blk,),
        in_specs=[pl.BlockSpec((_D, _TBLK), lambda i: (0, i))],
        out_specs=pl.BlockSpec((_TBLK, _DP), lambda i: (i, 0)),
        out_shape=jax.ShapeDtypeStruct((V, _DP), jnp.float32),
    )


@functools.lru_cache(maxsize=None)
def _build(B):
    b_per_w = B // _NW
    n_chunks = b_per_w // _CHUNK

    @functools.partial(
        pl.kernel,
        mesh=_mesh,
        compiler_params=pltpu.CompilerParams(use_tc_tiling_on_sc=True),
        out_type=jax.ShapeDtypeStruct((B, _DP), jnp.float32),
        scratch_types=[
            pltpu.VMEM((b_per_w,), jnp.int32),
            pltpu.VMEM((_NBUF, _CHUNK, _DP), jnp.float32),
            pltpu.SemaphoreType.DMA,
            pltpu.SemaphoreType.DMA,
            pltpu.SemaphoreType.DMA,
            pltpu.SemaphoreType.DMA,
        ],
    )
    def emb(idx_hbm, table_hbm, out_hbm, idx_v, rows_v, g0, g1, o0, o1):
        gsem = (g0, g1)
        osem = (o0, o1)
        wid = lax.axis_index("s") * 2 + lax.axis_index("c")
        base = wid * b_per_w
        pltpu.sync_copy(idx_hbm.at[pl.ds(base, b_per_w)], idx_v)

        gathers = [None] * _NBUF
        outs = [None] * _NBUF
        for i in range(n_chunks):
            b = i % _NBUF
            if outs[b] is not None:
                outs[b].wait()
            gathers[b] = pltpu.async_copy(
                table_hbm.at[idx_v.at[pl.ds(i * _CHUNK, _CHUNK)]],
                rows_v.at[b], gsem[b])
            if i > 0:
                pb = (i - 1) % _NBUF
                gathers[pb].wait()
                outs[pb] = pltpu.async_copy(
                    rows_v.at[pb],
                    out_hbm.at[pl.ds(base + (i - 1) * _CHUNK, _CHUNK)],
                    osem[pb])
        last = n_chunks - 1
        lb = last % _NBUF
        gathers[lb].wait()
        outs[lb] = pltpu.async_copy(
            rows_v.at[lb],
            out_hbm.at[pl.ds(base + last * _CHUNK, _CHUNK)],
            osem[lb])
        for b in range(_NBUF):
            if outs[b] is not None:
                outs[b].wait()

    return emb


def kernel(input_ids, table):
    bt, h = input_ids.shape
    flat = input_ids.T.reshape(bt * h).astype(jnp.int32)
    tpad = _build_tr(table.shape[0])(table.T)
    out = _build(bt * h)(flat, tpad)
    return out[:, :_D].reshape(h, bt, _D).transpose(1, 0, 2)


# speedup vs baseline: 1.9631x; 1.1854x over previous
import functools

import jax
import jax.numpy as jnp
from jax import lax
from jax.experimental import pallas as pl
from jax.experimental.pallas import tpu as pltpu
from jax.experimental.pallas import tpu_sc as plsc

_D = 64
_DP = 128
_NW = 32
_CHUNK = 400
_NBUF = 2
_TBLK = 8192

_mesh = plsc.VectorSubcoreMesh(core_axis_name="c", subcore_axis_name="s")


@functools.lru_cache(maxsize=None)
def _build_tr(V):
    n_blk = (V + _TBLK - 1) // _TBLK

    def tk(t_ref, o_ref):
        o_ref[:, :_D] = t_ref[...].T

    return pl.pallas_call(
        tk,
        grid=(n_blk,),
        in_specs=[pl.BlockSpec((_D, _TBLK), lambda i: (0, i))],
        out_specs=pl.BlockSpec((_TBLK, _DP), lambda i: (i, 0)),
        out_shape=jax.ShapeDtypeStruct((V, _DP), jnp.float32),
    )


@functools.lru_cache(maxsize=None)
def _build(B):
    b_per_w = B // _NW
    n_chunks = b_per_w // _CHUNK

    @functools.partial(
        pl.kernel,
        mesh=_mesh,
        compiler_params=pltpu.CompilerParams(use_tc_tiling_on_sc=True),
        out_type=jax.ShapeDtypeStruct((B, _DP), jnp.float32),
        scratch_types=[
            pltpu.VMEM((b_per_w,), jnp.int32),
            pltpu.VMEM((_NBUF, _CHUNK, _DP), jnp.float32),
            pltpu.SemaphoreType.DMA,
            pltpu.SemaphoreType.DMA,
            pltpu.SemaphoreType.DMA,
            pltpu.SemaphoreType.DMA,
        ],
    )
    def emb(idx_hbm, table_hbm, out_hbm, idx_v, rows_v, g0, g1, o0, o1):
        gsem = (g0, g1)
        osem = (o0, o1)
        wid = lax.axis_index("s") * 2 + lax.axis_index("c")
        base = wid * b_per_w
        pltpu.sync_copy(idx_hbm.at[pl.ds(base, b_per_w)], idx_v)

        gathers = [None] * _NBUF
        outs = [None] * _NBUF
        for i in range(n_chunks):
            b = i % _NBUF
            if outs[b] is not None:
                outs[b].wait()
            gathers[b] = pltpu.async_copy(
                table_hbm.at[idx_v.at[pl.ds(i * _CHUNK, _CHUNK)]],
                rows_v.at[b], gsem[b])
            if i > 0:
                pb = (i - 1) % _NBUF
                gathers[pb].wait()
                outs[pb] = pltpu.async_copy(
                    rows_v.at[pb],
                    out_hbm.at[pl.ds(base + (i - 1) * _CHUNK, _CHUNK)],
                    osem[pb])
        last = n_chunks - 1
        lb = last % _NBUF
        gathers[lb].wait()
        outs[lb] = pltpu.async_copy(
            rows_v.at[lb],
            out_hbm.at[pl.ds(base + last * _CHUNK, _CHUNK)],
            osem[lb])
        for b in range(_NBUF):
            if outs[b] is not None:
                outs[b].wait()

    return emb


def kernel(input_ids, table):
    bt, h = input_ids.shape
    flat = input_ids.T.reshape(bt * h).astype(jnp.int32)
    tpad = _build_tr(table.shape[0])(table.T)
    out = _build(bt * h)(flat, tpad)
    return out[:, :_D].reshape(h, bt, _D).transpose(1, 0, 2)


# speedup vs baseline: 2.0632x; 1.0510x over previous
import functools

import jax
import jax.numpy as jnp
from jax import lax
from jax.experimental import pallas as pl
from jax.experimental.pallas import tpu as pltpu
from jax.experimental.pallas import tpu_sc as plsc

_D = 64
_DP = 128
_NW = 32
_CHUNK = 400
_NBUF = 2
_TBLK = 16384

_mesh = plsc.VectorSubcoreMesh(core_axis_name="c", subcore_axis_name="s")


@functools.lru_cache(maxsize=None)
def _build_tr(V):
    n_blk = (V + _TBLK - 1) // _TBLK

    def tk(t_ref, o_ref):
        o_ref[:, :_D] = t_ref[...].T

    return pl.pallas_call(
        tk,
        grid=(n_blk,),
        in_specs=[pl.BlockSpec((_D, _TBLK), lambda i: (0, i))],
        out_specs=pl.BlockSpec((_TBLK, _DP), lambda i: (i, 0)),
        out_shape=jax.ShapeDtypeStruct((V, _DP), jnp.float32),
    )


@functools.lru_cache(maxsize=None)
def _build(B):
    b_per_w = B // _NW
    n_chunks = b_per_w // _CHUNK

    @functools.partial(
        pl.kernel,
        mesh=_mesh,
        compiler_params=pltpu.CompilerParams(use_tc_tiling_on_sc=True),
        out_type=jax.ShapeDtypeStruct((B, _DP), jnp.float32),
        scratch_types=[
            pltpu.VMEM((b_per_w,), jnp.int32),
            pltpu.VMEM((_NBUF, _CHUNK, _DP), jnp.float32),
            pltpu.SemaphoreType.DMA,
            pltpu.SemaphoreType.DMA,
            pltpu.SemaphoreType.DMA,
            pltpu.SemaphoreType.DMA,
        ],
    )
    def emb(idx_hbm, table_hbm, out_hbm, idx_v, rows_v, g0, g1, o0, o1):
        gsem = (g0, g1)
        osem = (o0, o1)
        wid = lax.axis_index("s") * 2 + lax.axis_index("c")
        base = wid * b_per_w
        pltpu.sync_copy(idx_hbm.at[pl.ds(base, b_per_w)], idx_v)

        gathers = [None] * _NBUF
        outs = [None] * _NBUF
        for i in range(n_chunks):
            b = i % _NBUF
            if outs[b] is not None:
                outs[b].wait()
            gathers[b] = pltpu.async_copy(
                table_hbm.at[idx_v.at[pl.ds(i * _CHUNK, _CHUNK)]],
                rows_v.at[b], gsem[b])
            if i > 0:
                pb = (i - 1) % _NBUF
                gathers[pb].wait()
                outs[pb] = pltpu.async_copy(
                    rows_v.at[pb],
                    out_hbm.at[pl.ds(base + (i - 1) * _CHUNK, _CHUNK)],
                    osem[pb])
        last = n_chunks - 1
        lb = last % _NBUF
        gathers[lb].wait()
        outs[lb] = pltpu.async_copy(
            rows_v.at[lb],
            out_hbm.at[pl.ds(base + last * _CHUNK, _CHUNK)],
            osem[lb])
        for b in range(_NBUF):
            if outs[b] is not None:
                outs[b].wait()

    return emb


def kernel(input_ids, table):
    bt, h = input_ids.shape
    flat = input_ids.T.reshape(bt * h).astype(jnp.int32)
    tpad = _build_tr(table.shape[0])(table.T)
    out = _build(bt * h)(flat, tpad)
    return out[:, :_D].reshape(h, bt, _D).transpose(1, 0, 2)


# speedup vs baseline: 2.0967x; 1.0163x over previous
import functools

import jax
import jax.numpy as jnp
from jax import lax
from jax.experimental import pallas as pl
from jax.experimental.pallas import tpu as pltpu
from jax.experimental.pallas import tpu_sc as plsc

_D = 64
_DP = 128
_NW = 32
_CHUNK = 400
_NBUF = 2
_TBLK = 32768

_mesh = plsc.VectorSubcoreMesh(core_axis_name="c", subcore_axis_name="s")


@functools.lru_cache(maxsize=None)
def _build_tr(V):
    n_blk = (V + _TBLK - 1) // _TBLK

    def tk(t_ref, o_ref):
        o_ref[:, :_D] = t_ref[...].T

    return pl.pallas_call(
        tk,
        grid=(n_blk,),
        in_specs=[pl.BlockSpec((_D, _TBLK), lambda i: (0, i))],
        out_specs=pl.BlockSpec((_TBLK, _DP), lambda i: (i, 0)),
        out_shape=jax.ShapeDtypeStruct((V, _DP), jnp.float32),
    )


@functools.lru_cache(maxsize=None)
def _build(B):
    b_per_w = B // _NW
    n_chunks = b_per_w // _CHUNK

    @functools.partial(
        pl.kernel,
        mesh=_mesh,
        compiler_params=pltpu.CompilerParams(use_tc_tiling_on_sc=True),
        out_type=jax.ShapeDtypeStruct((B, _DP), jnp.float32),
        scratch_types=[
            pltpu.VMEM((b_per_w,), jnp.int32),
            pltpu.VMEM((_NBUF, _CHUNK, _DP), jnp.float32),
            pltpu.SemaphoreType.DMA,
            pltpu.SemaphoreType.DMA,
            pltpu.SemaphoreType.DMA,
            pltpu.SemaphoreType.DMA,
        ],
    )
    def emb(idx_hbm, table_hbm, out_hbm, idx_v, rows_v, g0, g1, o0, o1):
        gsem = (g0, g1)
        osem = (o0, o1)
        wid = lax.axis_index("s") * 2 + lax.axis_index("c")
        base = wid * b_per_w
        pltpu.sync_copy(idx_hbm.at[pl.ds(base, b_per_w)], idx_v)

        gathers = [None] * _NBUF
        outs = [None] * _NBUF
        for i in range(n_chunks):
            b = i % _NBUF
            if outs[b] is not None:
                outs[b].wait()
            gathers[b] = pltpu.async_copy(
                table_hbm.at[idx_v.at[pl.ds(i * _CHUNK, _CHUNK)]],
                rows_v.at[b], gsem[b])
            if i > 0:
                pb = (i - 1) % _NBUF
                gathers[pb].wait()
                outs[pb] = pltpu.async_copy(
                    rows_v.at[pb],
                    out_hbm.at[pl.ds(base + (i - 1) * _CHUNK, _CHUNK)],
                    osem[pb])
        last = n_chunks - 1
        lb = last % _NBUF
        gathers[lb].wait()
        outs[lb] = pltpu.async_copy(
            rows_v.at[lb],
            out_hbm.at[pl.ds(base + last * _CHUNK, _CHUNK)],
            osem[lb])
        for b in range(_NBUF):
            if outs[b] is not None:
                outs[b].wait()

    return emb


def kernel(input_ids, table):
    bt, h = input_ids.shape
    flat = input_ids.T.reshape(bt * h).astype(jnp.int32)
    tpad = _build_tr(table.shape[0])(table.T)
    out = _build(bt * h)(flat, tpad)
    return out[:, :_D].reshape(h, bt, _D).transpose(1, 0, 2)


# speedup vs baseline: 2.3364x; 1.1143x over previous
import functools

import jax
import jax.numpy as jnp
from jax import lax
from jax.experimental import pallas as pl
from jax.experimental.pallas import tpu as pltpu
from jax.experimental.pallas import tpu_sc as plsc

_D = 64
_DP = 128
_NW = 32
_CHUNK = 320
_NBUF = 3
_TBLK = 32768

_mesh = plsc.VectorSubcoreMesh(core_axis_name="c", subcore_axis_name="s")


@functools.lru_cache(maxsize=None)
def _build_tr(V):
    n_blk = (V + _TBLK - 1) // _TBLK

    def tk(t_ref, o_ref):
        o_ref[:, :_D] = t_ref[...].T

    return pl.pallas_call(
        tk,
        grid=(n_blk,),
        in_specs=[pl.BlockSpec((_D, _TBLK), lambda i: (0, i))],
        out_specs=pl.BlockSpec((_TBLK, _DP), lambda i: (i, 0)),
        out_shape=jax.ShapeDtypeStruct((V, _DP), jnp.float32),
    )


@functools.lru_cache(maxsize=None)
def _build(B):
    b_per_w = B // _NW
    n_chunks = b_per_w // _CHUNK

    @functools.partial(
        pl.kernel,
        mesh=_mesh,
        compiler_params=pltpu.CompilerParams(use_tc_tiling_on_sc=True),
        out_type=jax.ShapeDtypeStruct((B, _DP), jnp.float32),
        scratch_types=[
            pltpu.VMEM((b_per_w,), jnp.int32),
            pltpu.VMEM((_NBUF, _CHUNK, _DP), jnp.float32),
            pltpu.SemaphoreType.DMA,
            pltpu.SemaphoreType.DMA,
            pltpu.SemaphoreType.DMA,
            pltpu.SemaphoreType.DMA,
            pltpu.SemaphoreType.DMA,
            pltpu.SemaphoreType.DMA,
        ],
    )
    def emb(idx_hbm, table_hbm, out_hbm, idx_v, rows_v,
            g0, g1, g2, o0, o1, o2):
        gsem = (g0, g1, g2)
        osem = (o0, o1, o2)
        wid = lax.axis_index("s") * 2 + lax.axis_index("c")
        base = wid * b_per_w
        pltpu.sync_copy(idx_hbm.at[pl.ds(base, b_per_w)], idx_v)

        gathers = [None] * _NBUF
        outs = [None] * _NBUF
        for i in range(n_chunks):
            b = i % _NBUF
            if outs[b] is not None:
                outs[b].wait()
            gathers[b] = pltpu.async_copy(
                table_hbm.at[idx_v.at[pl.ds(i * _CHUNK, _CHUNK)]],
                rows_v.at[b], gsem[b])
            if i > 0:
                pb = (i - 1) % _NBUF
                gathers[pb].wait()
                outs[pb] = pltpu.async_copy(
                    rows_v.at[pb],
                    out_hbm.at[pl.ds(base + (i - 1) * _CHUNK, _CHUNK)],
                    osem[pb])
        last = n_chunks - 1
        lb = last % _NBUF
        gathers[lb].wait()
        outs[lb] = pltpu.async_copy(
            rows_v.at[lb],
            out_hbm.at[pl.ds(base + last * _CHUNK, _CHUNK)],
            osem[lb])
        for b in range(_NBUF):
            if outs[b] is not None:
                outs[b].wait()

    return emb


def kernel(input_ids, table):
    bt, h = input_ids.shape
    flat = input_ids.T.reshape(bt * h).astype(jnp.int32)
    tpad = _build_tr(table.shape[0])(table.T)
    out = _build(bt * h)(flat, tpad)
    return out[:, :_D].reshape(h, bt, _D).transpose(1, 0, 2)
